# Initial kernel scaffold; baseline (speedup 1.0000x reference)
import jax
import jax.numpy as jnp
from jax.experimental import pallas as pl


def kernel(x, S, H, T, LUT):
    raise NotImplementedError("write your pallas kernel here")


# speedup vs baseline: 1.0460x; 1.0460x over previous
import functools

import jax
import jax.numpy as jnp
from jax import lax
from jax.experimental import pallas as pl
from jax.experimental.pallas import tpu as pltpu
from jax.experimental.pallas import tpu_sc as plsc


_BLK = 512
_K = 4096
_C = 5


def _tc_body(x_ref, w_ref, b_ref, h_ref, out_ref):
    c = pl.program_id(1)
    xb = x_ref[...]
    w = w_ref[0]
    y = jnp.dot(xb, w, preferred_element_type=jnp.float32) - b_ref[0]
    s = jnp.sign(y)
    logits = jnp.dot(s, h_ref[...], preferred_element_type=jnp.float32)
    m = jnp.max(logits, axis=1, keepdims=True)
    ii = lax.broadcasted_iota(jnp.int32, logits.shape, 1)
    idx = jnp.min(jnp.where(logits == m, ii, _K), axis=1)
    out_ref[0, 0, :] = idx + c * _K


def _tc_indices(x, w5, b5, h):
    n = x.shape[0]
    grid = (n // _BLK, _C)
    return pl.pallas_call(
        _tc_body,
        grid=grid,
        in_specs=[
            pl.BlockSpec((_BLK, 30), lambda nb, c: (nb, 0)),
            pl.BlockSpec((1, 30, 45), lambda nb, c: (c, 0, 0)),
            pl.BlockSpec((1, 1, 45), lambda nb, c: (c, 0, 0)),
            pl.BlockSpec((45, _K), lambda nb, c: (0, 0)),
        ],
        out_specs=pl.BlockSpec((1, 1, _BLK), lambda nb, c: (c, 0, nb)),
        out_shape=jax.ShapeDtypeStruct((_C, 1, n), jnp.int32),
    )(x, w5, b5, h)


_DP = 128
_CHUNK = 128


def _make_sc_gather(total):
    info = plsc.get_sparse_core_info()
    nw = info.num_cores * info.num_subcores
    b_per_w = total // nw
    n_chunks = b_per_w // _CHUNK

    mesh = plsc.VectorSubcoreMesh(core_axis_name="c", subcore_axis_name="s")

    @functools.partial(
        pl.kernel,
        mesh=mesh,
        out_type=jax.ShapeDtypeStruct((total, _DP), jnp.float32),
        scratch_types=[
            pltpu.VMEM((n_chunks, _CHUNK), jnp.int32),
            pltpu.VMEM((_CHUNK, _DP), jnp.float32),
            pltpu.SemaphoreType.DMA,
        ],
    )
    def gather_k(idx_hbm, table_hbm, out_hbm, idx_v, rows_v, sem):
        wid = lax.axis_index("s") * info.num_cores + lax.axis_index("c")
        base = wid * b_per_w
        pltpu.sync_copy(idx_hbm.at[wid], idx_v)
        for j in range(n_chunks):
            pltpu.async_copy(table_hbm.at[idx_v.at[j]], rows_v, sem).wait()
            pltpu.sync_copy(rows_v, out_hbm.at[pl.ds(base + j * _CHUNK, _CHUNK)])

    return gather_k


def kernel(x, S, H, T, LUT):
    n = x.shape[0]
    wbig = jnp.einsum('jdk,ji->jdik', S, jnp.eye(15, dtype=S.dtype))
    w5 = wbig.reshape(30, 5, 45).transpose(1, 0, 2)
    b5 = (T.reshape(225) + jnp.float32(0.0001)).reshape(5, 1, 45)

    idx = _tc_indices(x, w5, b5, H)
    total = n * _C
    info = plsc.get_sparse_core_info()
    nw = info.num_cores * info.num_subcores
    flat_idx = idx.reshape(_C, n).T.reshape(nw, -1, _CHUNK)

    d = LUT.shape[-1]
    lut_pad = jnp.pad(LUT.reshape(_C * _K, d), ((0, 0), (0, _DP - d)))
    rows = _make_sc_gather(total)(flat_idx, lut_pad)
    return rows[:, :d].reshape(n, _C, d)


# speedup vs baseline: 1.1801x; 1.1282x over previous
import functools

import jax
import jax.numpy as jnp
from jax import lax
from jax.experimental import pallas as pl
from jax.experimental.pallas import tpu as pltpu
from jax.experimental.pallas import tpu_sc as plsc


_BLK = 512
_K = 4096
_C = 5


def _tc_body(x_ref, w_ref, b_ref, h_ref, out_ref):
    c = pl.program_id(1)
    xb = x_ref[...]
    w = w_ref[0]
    y = jnp.dot(xb, w, preferred_element_type=jnp.float32) - b_ref[0]
    s = jnp.sign(y)
    logits = jnp.dot(s, h_ref[...], preferred_element_type=jnp.float32)
    m = jnp.max(logits, axis=1, keepdims=True)
    ii_rev = (_K - 1) - lax.broadcasted_iota(jnp.int32, logits.shape, 1)
    rev = jnp.max(jnp.where(logits >= m, ii_rev.astype(jnp.float32), -1.0), axis=1)
    idx = (_K - 1) - rev.astype(jnp.int32)
    out_ref[0, 0, :] = idx + c * _K


def _tc_indices(x, w5, b5, h):
    n = x.shape[0]
    grid = (n // _BLK, _C)
    return pl.pallas_call(
        _tc_body,
        grid=grid,
        in_specs=[
            pl.BlockSpec((_BLK, 30), lambda nb, c: (nb, 0)),
            pl.BlockSpec((1, 30, 45), lambda nb, c: (c, 0, 0)),
            pl.BlockSpec((1, 1, 45), lambda nb, c: (c, 0, 0)),
            pl.BlockSpec((45, _K), lambda nb, c: (0, 0)),
        ],
        out_specs=pl.BlockSpec((1, 1, _BLK), lambda nb, c: (c, 0, nb)),
        out_shape=jax.ShapeDtypeStruct((_C, 1, n), jnp.int32),
    )(x, w5, b5, h)


_DP = 128
_CHUNK = 128


def _make_sc_gather(total):
    info = plsc.get_sparse_core_info()
    nw = info.num_cores * info.num_subcores
    b_per_w = total // nw
    n_chunks = b_per_w // _CHUNK

    mesh = plsc.VectorSubcoreMesh(core_axis_name="c", subcore_axis_name="s")

    @functools.partial(
        pl.kernel,
        mesh=mesh,
        out_type=jax.ShapeDtypeStruct((total, _DP), jnp.float32),
        scratch_types=[
            pltpu.VMEM((n_chunks, _CHUNK), jnp.int32),
            pltpu.VMEM((_CHUNK, _DP), jnp.float32),
            pltpu.SemaphoreType.DMA,
        ],
    )
    def gather_k(idx_hbm, table_hbm, out_hbm, idx_v, rows_v, sem):
        wid = lax.axis_index("s") * info.num_cores + lax.axis_index("c")
        base = wid * b_per_w
        pltpu.sync_copy(idx_hbm.at[wid], idx_v)
        for j in range(n_chunks):
            pltpu.async_copy(table_hbm.at[idx_v.at[j]], rows_v, sem).wait()
            pltpu.sync_copy(rows_v, out_hbm.at[pl.ds(base + j * _CHUNK, _CHUNK)])

    return gather_k


def kernel(x, S, H, T, LUT):
    n = x.shape[0]
    wbig = jnp.einsum('jdk,ji->jdik', S, jnp.eye(15, dtype=S.dtype))
    w5 = wbig.reshape(30, 5, 45).transpose(1, 0, 2)
    b5 = (T.reshape(225) + jnp.float32(0.0001)).reshape(5, 1, 45)

    idx = _tc_indices(x, w5, b5, H)
    total = n * _C
    info = plsc.get_sparse_core_info()
    nw = info.num_cores * info.num_subcores
    flat_idx = idx.reshape(_C, n).T.reshape(nw, -1, _CHUNK)

    d = LUT.shape[-1]
    lut_pad = jnp.pad(LUT.reshape(_C * _K, d), ((0, 0), (0, _DP - d)))
    rows = _make_sc_gather(total)(flat_idx, lut_pad)
    return rows[:, :d].reshape(n, _C, d)


# speedup vs baseline: 1.2825x; 1.0868x over previous
import functools

import jax
import jax.numpy as jnp
from jax import lax
from jax.experimental import pallas as pl
from jax.experimental.pallas import tpu as pltpu
from jax.experimental.pallas import tpu_sc as plsc


_BLK = 512
_K = 4096
_C = 5


def _tc_body(x_ref, w_ref, b_ref, h_ref, out_ref):
    c = pl.program_id(1)
    xb = x_ref[...]
    w = w_ref[0]
    y = jnp.dot(xb, w, preferred_element_type=jnp.float32) - b_ref[0]
    s = jnp.sign(y)
    logits = jnp.dot(s, h_ref[...], preferred_element_type=jnp.float32)
    m = jnp.max(logits, axis=1, keepdims=True)
    ii_rev = (_K - 1) - lax.broadcasted_iota(jnp.int32, logits.shape, 1)
    rev = jnp.max(jnp.where(logits >= m, ii_rev.astype(jnp.float32), -1.0), axis=1)
    idx = (_K - 1) - rev.astype(jnp.int32)
    out_ref[0, 0, :] = idx + c * _K


def _tc_indices(x, w5, b5, h):
    n = x.shape[0]
    grid = (n // _BLK, _C)
    return pl.pallas_call(
        _tc_body,
        grid=grid,
        in_specs=[
            pl.BlockSpec((_BLK, 30), lambda nb, c: (nb, 0)),
            pl.BlockSpec((1, 30, 45), lambda nb, c: (c, 0, 0)),
            pl.BlockSpec((1, 1, 45), lambda nb, c: (c, 0, 0)),
            pl.BlockSpec((45, _K), lambda nb, c: (0, 0)),
        ],
        out_specs=pl.BlockSpec((1, 1, _BLK), lambda nb, c: (c, 0, nb)),
        out_shape=jax.ShapeDtypeStruct((_C, 1, n), jnp.int32),
    )(x, w5, b5, h)


_D = 32
_DP = 128
_CHUNK = 128


def _make_sc_gather(total):
    info = plsc.get_sparse_core_info()
    nw = info.num_cores * info.num_subcores
    b_per_w = total // nw
    n_chunks = b_per_w // _CHUNK

    mesh = plsc.VectorSubcoreMesh(core_axis_name="c", subcore_axis_name="s")

    grp = 4
    n_grps = n_chunks // grp

    @functools.partial(
        pl.kernel,
        mesh=mesh,
        out_type=jax.ShapeDtypeStruct((total * _D,), jnp.float32),
        scratch_types=[
            pltpu.VMEM((n_chunks, _CHUNK), jnp.int32),
            pltpu.VMEM((2, _CHUNK, _DP), jnp.float32),
            pltpu.VMEM((grp * _CHUNK * _D,), jnp.float32),
            pltpu.SemaphoreType.DMA,
            pltpu.SemaphoreType.DMA,
        ],
    )
    def gather_k(idx_hbm, table_hbm, out_hbm, idx_v, pbuf, cbuf, sem0, sem1):
        wid = lax.axis_index("s") * info.num_cores + lax.axis_index("c")
        base = wid * b_per_w
        sems = (sem0, sem1)
        pltpu.sync_copy(idx_hbm.at[wid], idx_v)
        pltpu.async_copy(table_hbm.at[idx_v.at[0]], pbuf.at[0], sem0)

        def compact(buf_i, u):
            def crow(r4, _):
                for uu in range(4):
                    r = r4 * 4 + uu
                    dst = (u * _CHUNK + r) * _D
                    cbuf[pl.ds(dst, 16)] = pbuf[buf_i, r, pl.ds(0, 16)]
                    cbuf[pl.ds(dst + 16, 16)] = pbuf[buf_i, r, pl.ds(16, 16)]
                return 0

            lax.fori_loop(0, _CHUNK // 4, crow, 0)

        def body(g, _):
            for u in range(grp):
                j = g * grp + u
                bi = u % 2

                @pl.when(j + 1 < n_chunks)
                def _():
                    pltpu.async_copy(table_hbm.at[idx_v.at[j + 1]],
                                     pbuf.at[1 - bi], sems[1 - bi])

                pltpu.make_async_copy(table_hbm.at[idx_v.at[0]],
                                      pbuf.at[bi], sems[bi]).wait()
                compact(bi, u)
            pltpu.sync_copy(
                cbuf,
                out_hbm.at[pl.ds((base + g * grp * _CHUNK) * _D,
                                 grp * _CHUNK * _D)])
            return 0

        lax.fori_loop(0, n_grps, body, 0)

    return gather_k


def kernel(x, S, H, T, LUT):
    n = x.shape[0]
    wbig = jnp.einsum('jdk,ji->jdik', S, jnp.eye(15, dtype=S.dtype))
    w5 = wbig.reshape(30, 5, 45).transpose(1, 0, 2)
    b5 = (T.reshape(225) + jnp.float32(0.0001)).reshape(5, 1, 45)

    idx = _tc_indices(x, w5, b5, H)
    total = n * _C
    info = plsc.get_sparse_core_info()
    nw = info.num_cores * info.num_subcores
    flat_idx = idx.reshape(_C, n).T.reshape(nw, -1, _CHUNK)

    lut_pad = jnp.pad(LUT.reshape(_C * _K, _D), ((0, 0), (0, _DP - _D)))
    rows = _make_sc_gather(total)(flat_idx, lut_pad)
    return rows.reshape(n, _C, _D)


# speedup vs baseline: 1.5312x; 1.1940x over previous
import functools

import jax
import jax.numpy as jnp
from jax import lax
from jax.experimental import pallas as pl
from jax.experimental.pallas import tpu as pltpu
from jax.experimental.pallas import tpu_sc as plsc


_BLK = 512
_K = 4096
_C = 5


def _tc_body(x_ref, w_ref, b_ref, h_ref, out_ref):
    c = pl.program_id(1)
    xb = x_ref[...]
    w = w_ref[0]
    y = jnp.dot(xb, w, preferred_element_type=jnp.float32) - b_ref[0]
    s = jnp.sign(y)
    logits = jnp.dot(s, h_ref[...], preferred_element_type=jnp.float32)
    idx = jnp.argmax(logits, axis=1).astype(jnp.int32)
    out_ref[0, 0, :] = idx + c * _K


def _tc_indices(x, w5, b5, h):
    n = x.shape[0]
    grid = (n // _BLK, _C)
    return pl.pallas_call(
        _tc_body,
        grid=grid,
        in_specs=[
            pl.BlockSpec((_BLK, 30), lambda nb, c: (nb, 0)),
            pl.BlockSpec((1, 30, 45), lambda nb, c: (c, 0, 0)),
            pl.BlockSpec((1, 1, 45), lambda nb, c: (c, 0, 0)),
            pl.BlockSpec((45, _K), lambda nb, c: (0, 0)),
        ],
        out_specs=pl.BlockSpec((1, 1, _BLK), lambda nb, c: (c, 0, nb)),
        out_shape=jax.ShapeDtypeStruct((_C, 1, n), jnp.int32),
    )(x, w5, b5, h)


_D = 32
_DP = 128
_CHUNK = 128


def _make_sc_gather(total):
    info = plsc.get_sparse_core_info()
    nw = info.num_cores * info.num_subcores
    b_per_w = total // nw
    n_chunks = b_per_w // _CHUNK

    mesh = plsc.VectorSubcoreMesh(core_axis_name="c", subcore_axis_name="s")

    grp = 4
    n_grps = n_chunks // grp

    @functools.partial(
        pl.kernel,
        mesh=mesh,
        out_type=jax.ShapeDtypeStruct((total * _D,), jnp.float32),
        scratch_types=[
            pltpu.VMEM((n_chunks, _CHUNK), jnp.int32),
            pltpu.VMEM((2, _CHUNK, _DP), jnp.float32),
            pltpu.VMEM((grp * _CHUNK * _D,), jnp.float32),
            pltpu.SemaphoreType.DMA,
            pltpu.SemaphoreType.DMA,
        ],
    )
    def gather_k(idx_hbm, table_hbm, out_hbm, idx_v, pbuf, cbuf, sem0, sem1):
        wid = lax.axis_index("s") * info.num_cores + lax.axis_index("c")
        base = wid * b_per_w
        sems = (sem0, sem1)
        pltpu.sync_copy(idx_hbm.at[wid], idx_v)
        pltpu.async_copy(table_hbm.at[idx_v.at[0]], pbuf.at[0], sem0)

        def compact(buf_i, u):
            def crow(r4, _):
                for uu in range(4):
                    r = r4 * 4 + uu
                    dst = (u * _CHUNK + r) * _D
                    cbuf[pl.ds(dst, 16)] = pbuf[buf_i, r, pl.ds(0, 16)]
                    cbuf[pl.ds(dst + 16, 16)] = pbuf[buf_i, r, pl.ds(16, 16)]
                return 0

            lax.fori_loop(0, _CHUNK // 4, crow, 0)

        def body(g, _):
            for u in range(grp):
                j = g * grp + u
                bi = u % 2

                @pl.when(j + 1 < n_chunks)
                def _():
                    pltpu.async_copy(table_hbm.at[idx_v.at[j + 1]],
                                     pbuf.at[1 - bi], sems[1 - bi])

                pltpu.make_async_copy(table_hbm.at[idx_v.at[0]],
                                      pbuf.at[bi], sems[bi]).wait()
                compact(bi, u)
            pltpu.sync_copy(
                cbuf,
                out_hbm.at[pl.ds((base + g * grp * _CHUNK) * _D,
                                 grp * _CHUNK * _D)])
            return 0

        lax.fori_loop(0, n_grps, body, 0)

    return gather_k


def kernel(x, S, H, T, LUT):
    n = x.shape[0]
    wbig = jnp.einsum('jdk,ji->jdik', S, jnp.eye(15, dtype=S.dtype))
    w5 = wbig.reshape(30, 5, 45).transpose(1, 0, 2)
    b5 = (T.reshape(225) + jnp.float32(0.0001)).reshape(5, 1, 45)

    idx = _tc_indices(x, w5, b5, H)
    total = n * _C
    info = plsc.get_sparse_core_info()
    nw = info.num_cores * info.num_subcores
    flat_idx = idx.reshape(_C, n).T.reshape(nw, -1, _CHUNK)

    lut_pad = jnp.pad(LUT.reshape(_C * _K, _D), ((0, 0), (0, _DP - _D)))
    rows = _make_sc_gather(total)(flat_idx, lut_pad)
    return rows.reshape(n, _C, _D)


# speedup vs baseline: 1.5793x; 1.0314x over previous
import functools

import jax
import jax.numpy as jnp
from jax import lax
from jax.experimental import pallas as pl
from jax.experimental.pallas import tpu as pltpu
from jax.experimental.pallas import tpu_sc as plsc


_BLK = 1024
_K = 4096
_C = 5


def _tc_body(x_ref, w_ref, b_ref, h_ref, out_ref):
    c = pl.program_id(1)
    xb = x_ref[...]
    w = w_ref[0]
    y = jnp.dot(xb, w, preferred_element_type=jnp.float32) - b_ref[0]
    s = jnp.sign(y)
    logits = jnp.dot(s, h_ref[...], preferred_element_type=jnp.float32)
    idx = jnp.argmax(logits, axis=1).astype(jnp.int32)
    out_ref[0, 0, :] = idx + c * _K


def _tc_indices(x, w5, b5, h):
    n = x.shape[0]
    grid = (n // _BLK, _C)
    return pl.pallas_call(
        _tc_body,
        grid=grid,
        in_specs=[
            pl.BlockSpec((_BLK, 30), lambda nb, c: (nb, 0)),
            pl.BlockSpec((1, 30, 45), lambda nb, c: (c, 0, 0)),
            pl.BlockSpec((1, 1, 45), lambda nb, c: (c, 0, 0)),
            pl.BlockSpec((45, _K), lambda nb, c: (0, 0)),
        ],
        out_specs=pl.BlockSpec((1, 1, _BLK), lambda nb, c: (c, 0, nb)),
        out_shape=jax.ShapeDtypeStruct((_C, 1, n), jnp.int32),
    )(x, w5, b5, h)


_D = 32
_DP = 128
_CHUNK = 128


def _make_sc_gather(total):
    info = plsc.get_sparse_core_info()
    nw = info.num_cores * info.num_subcores
    b_per_w = total // nw
    n_chunks = b_per_w // _CHUNK

    mesh = plsc.VectorSubcoreMesh(core_axis_name="c", subcore_axis_name="s")

    grp = 4
    n_grps = n_chunks // grp

    @functools.partial(
        pl.kernel,
        mesh=mesh,
        out_type=jax.ShapeDtypeStruct((total * _D,), jnp.float32),
        scratch_types=[
            pltpu.VMEM((n_chunks, _CHUNK), jnp.int32),
            pltpu.VMEM((2, _CHUNK, _DP), jnp.float32),
            pltpu.VMEM((grp * _CHUNK * _D,), jnp.float32),
            pltpu.SemaphoreType.DMA,
            pltpu.SemaphoreType.DMA,
        ],
    )
    def gather_k(idx_hbm, table_hbm, out_hbm, idx_v, pbuf, cbuf, sem0, sem1):
        wid = lax.axis_index("s") * info.num_cores + lax.axis_index("c")
        base = wid * b_per_w
        sems = (sem0, sem1)
        pltpu.sync_copy(idx_hbm.at[wid], idx_v)
        pltpu.async_copy(table_hbm.at[idx_v.at[0]], pbuf.at[0], sem0)

        def compact(buf_i, u):
            def crow(r4, _):
                for uu in range(4):
                    r = r4 * 4 + uu
                    dst = (u * _CHUNK + r) * _D
                    cbuf[pl.ds(dst, 16)] = pbuf[buf_i, r, pl.ds(0, 16)]
                    cbuf[pl.ds(dst + 16, 16)] = pbuf[buf_i, r, pl.ds(16, 16)]
                return 0

            lax.fori_loop(0, _CHUNK // 4, crow, 0)

        def body(g, _):
            for u in range(grp):
                j = g * grp + u
                bi = u % 2

                @pl.when(j + 1 < n_chunks)
                def _():
                    pltpu.async_copy(table_hbm.at[idx_v.at[j + 1]],
                                     pbuf.at[1 - bi], sems[1 - bi])

                pltpu.make_async_copy(table_hbm.at[idx_v.at[0]],
                                      pbuf.at[bi], sems[bi]).wait()
                compact(bi, u)
            pltpu.sync_copy(
                cbuf,
                out_hbm.at[pl.ds((base + g * grp * _CHUNK) * _D,
                                 grp * _CHUNK * _D)])
            return 0

        lax.fori_loop(0, n_grps, body, 0)

    return gather_k


def kernel(x, S, H, T, LUT):
    n = x.shape[0]
    wbig = jnp.einsum('jdk,ji->jdik', S, jnp.eye(15, dtype=S.dtype))
    w5 = wbig.reshape(30, 5, 45).transpose(1, 0, 2)
    b5 = (T.reshape(225) + jnp.float32(0.0001)).reshape(5, 1, 45)

    idx = _tc_indices(x, w5, b5, H)
    total = n * _C
    info = plsc.get_sparse_core_info()
    nw = info.num_cores * info.num_subcores
    flat_idx = idx.reshape(_C, n).T.reshape(nw, -1, _CHUNK)

    lut_pad = jnp.pad(LUT.reshape(_C * _K, _D), ((0, 0), (0, _DP - _D)))
    rows = _make_sc_gather(total)(flat_idx, lut_pad)
    return rows.reshape(n, _C, _D)


# speedup vs baseline: 1.6556x; 1.0483x over previous
import functools

import jax
import jax.numpy as jnp
from jax import lax
from jax.experimental import pallas as pl
from jax.experimental.pallas import tpu as pltpu
from jax.experimental.pallas import tpu_sc as plsc


_BLK = 1024
_K = 4096
_C = 5


def _tc_body(x_ref, w_ref, b_ref, h_ref, out_ref):
    c = pl.program_id(1)
    xb = x_ref[...]
    w = w_ref[0]
    y = jnp.dot(xb, w, preferred_element_type=jnp.float32) - b_ref[0]
    s = jnp.sign(y)
    logits = jnp.dot(s, h_ref[...], preferred_element_type=jnp.float32)
    idx = jnp.argmax(logits, axis=1).astype(jnp.int32)
    out_ref[0, 0, :] = idx + c * _K


def _tc_indices(x, w5, b5, h):
    n = x.shape[0]
    grid = (n // _BLK, _C)
    return pl.pallas_call(
        _tc_body,
        grid=grid,
        in_specs=[
            pl.BlockSpec((_BLK, 30), lambda nb, c: (nb, 0)),
            pl.BlockSpec((1, 30, 45), lambda nb, c: (c, 0, 0)),
            pl.BlockSpec((1, 1, 45), lambda nb, c: (c, 0, 0)),
            pl.BlockSpec((45, _K), lambda nb, c: (0, 0)),
        ],
        out_specs=pl.BlockSpec((1, 1, _BLK), lambda nb, c: (c, 0, nb)),
        out_shape=jax.ShapeDtypeStruct((_C, 1, n), jnp.int32),
    )(x, w5, b5, h)


_D = 32
_DP = 128
_CHUNK = 128


def _make_sc_gather(total):
    info = plsc.get_sparse_core_info()
    nw = info.num_cores * info.num_subcores
    b_per_w = total // nw
    n_chunks = b_per_w // _CHUNK

    mesh = plsc.VectorSubcoreMesh(core_axis_name="c", subcore_axis_name="s")

    grp = 4 if n_chunks % 4 == 0 else 2
    n_grps = n_chunks // grp

    @functools.partial(
        pl.kernel,
        mesh=mesh,
        out_type=jax.ShapeDtypeStruct((total * _D,), jnp.float32),
        scratch_types=[
            pltpu.VMEM((n_chunks, _CHUNK), jnp.int32),
            pltpu.VMEM((2, _CHUNK, _DP), jnp.float32),
            pltpu.VMEM((grp * _CHUNK * _D,), jnp.float32),
            pltpu.SemaphoreType.DMA,
            pltpu.SemaphoreType.DMA,
        ],
    )
    def gather_k(idx_hbm, table_hbm, out_hbm, idx_v, pbuf, cbuf, sem0, sem1):
        wid = lax.axis_index("s") * info.num_cores + lax.axis_index("c")
        base = wid * b_per_w
        sems = (sem0, sem1)
        pltpu.sync_copy(idx_hbm.at[wid], idx_v)
        pltpu.async_copy(table_hbm.at[idx_v.at[0]], pbuf.at[0], sem0)

        def compact(buf_i, u):
            def crow(r4, _):
                for uu in range(4):
                    r = r4 * 4 + uu
                    dst = (u * _CHUNK + r) * _D
                    cbuf[pl.ds(dst, 16)] = pbuf[buf_i, r, pl.ds(0, 16)]
                    cbuf[pl.ds(dst + 16, 16)] = pbuf[buf_i, r, pl.ds(16, 16)]
                return 0

            lax.fori_loop(0, _CHUNK // 4, crow, 0)

        def body(g, _):
            for u in range(grp):
                j = g * grp + u
                bi = u % 2

                @pl.when(j + 1 < n_chunks)
                def _():
                    pltpu.async_copy(table_hbm.at[idx_v.at[j + 1]],
                                     pbuf.at[1 - bi], sems[1 - bi])

                pltpu.make_async_copy(table_hbm.at[idx_v.at[0]],
                                      pbuf.at[bi], sems[bi]).wait()
                compact(bi, u)
            pltpu.sync_copy(
                cbuf,
                out_hbm.at[pl.ds((base + g * grp * _CHUNK) * _D,
                                 grp * _CHUNK * _D)])
            return 0

        lax.fori_loop(0, n_grps, body, 0)

    return gather_k


def kernel(x, S, H, T, LUT):
    n = x.shape[0]
    wbig = jnp.einsum('jdk,ji->jdik', S, jnp.eye(15, dtype=S.dtype))
    w5 = wbig.reshape(30, 5, 45).transpose(1, 0, 2)
    b5 = (T.reshape(225) + jnp.float32(0.0001)).reshape(5, 1, 45)

    lut_pad = jnp.pad(LUT.reshape(_C * _K, _D), ((0, 0), (0, _DP - _D)))
    info = plsc.get_sparse_core_info()
    nw = info.num_cores * info.num_subcores

    half = n // 2
    gather = _make_sc_gather(half * _C)
    outs = []
    for xh in (x[:half], x[half:]):
        idx = _tc_indices(xh, w5, b5, H)
        flat_idx = idx.reshape(_C, half).T.reshape(nw, -1, _CHUNK)
        outs.append(gather(flat_idx, lut_pad).reshape(half, _C, _D))
    return jnp.concatenate(outs, axis=0)
